# baseline (speedup 1.0000x reference)
import functools

import jax
import jax.numpy as jnp
from jax import lax
from jax.experimental import pallas as pl
from jax.experimental.pallas import tpu as pltpu
from jax.experimental.pallas import tpu_sc as plsc

BATCH = 16384
EMBED_DIM = 32
NUM_CORES = 2
NUM_SUBCORES = 16
LANES = 16
NUM_WORKERS = NUM_CORES * NUM_SUBCORES
B_PER_W = BATCH // NUM_WORKERS
GROUPS = B_PER_W // LANES


def _body(uid_hbm, iid_hbm, ut_hbm, it_hbm, out_hbm,
          uid_v, iid_v, urows, irows, out_v, sem_u, sem_i):
    wid = lax.axis_index("s") * NUM_CORES + lax.axis_index("c")
    base = wid * B_PER_W

    pltpu.sync_copy(uid_hbm.at[pl.ds(base, B_PER_W)], uid_v)
    pltpu.sync_copy(iid_hbm.at[pl.ds(base, B_PER_W)], iid_v)

    cp_u = pltpu.async_copy(ut_hbm.at[uid_v], urows, sem_u)
    cp_i = pltpu.async_copy(it_hbm.at[iid_v], irows, sem_i)
    cp_u.wait()
    cp_i.wait()

    lane = lax.iota(jnp.int32, LANES)

    def group(g, carry):
        r0 = g * LANES
        acc = jnp.zeros((LANES,), jnp.float32)
        for i in range(LANES):
            r = r0 + i
            u0 = urows[r, pl.ds(0, LANES)]
            u1 = urows[r, pl.ds(LANES, LANES)]
            v0 = irows[r, pl.ds(0, LANES)]
            v1 = irows[r, pl.ds(LANES, LANES)]
            p = u0 * v0 + u1 * v1
            acc = jnp.where(lane == i, jnp.sum(p), acc)
        out_v[pl.ds(r0, LANES)] = acc
        return carry

    lax.fori_loop(0, GROUPS, group, 0)

    pltpu.sync_copy(out_v, out_hbm.at[pl.ds(base, B_PER_W)])


@jax.jit
def kernel(user_ids, item_ids, user_table, item_table):
    mesh = plsc.VectorSubcoreMesh(
        core_axis_name="c", subcore_axis_name="s",
        num_cores=NUM_CORES, num_subcores=NUM_SUBCORES)
    f = pl.kernel(
        _body,
        out_type=jax.ShapeDtypeStruct((BATCH,), jnp.float32),
        mesh=mesh,
        compiler_params=pltpu.CompilerParams(
            needs_layout_passes=False, use_tc_tiling_on_sc=False),
        scratch_types=[
            pltpu.VMEM((B_PER_W,), jnp.int32),
            pltpu.VMEM((B_PER_W,), jnp.int32),
            pltpu.VMEM((B_PER_W, EMBED_DIM), jnp.float32),
            pltpu.VMEM((B_PER_W, EMBED_DIM), jnp.float32),
            pltpu.VMEM((B_PER_W,), jnp.float32),
            pltpu.SemaphoreType.DMA,
            pltpu.SemaphoreType.DMA,
        ],
    )
    return f(user_ids.astype(jnp.int32), item_ids.astype(jnp.int32),
             user_table, item_table)
